# Initial kernel scaffold; baseline (speedup 1.0000x reference)
import jax
import jax.numpy as jnp
from jax.experimental import pallas as pl


def kernel(x, edge_index, batch, W1_1, b1_1, W1_2, b1_2, gamma1, beta1, W2_1, b2_1, W2_2, b2_2, gamma2, beta2, W3_1, b3_1, W3_2, b3_2, gamma3, beta3):
    raise NotImplementedError("write your pallas kernel here")


# speedup vs baseline: 4.5149x; 4.5149x over previous
import functools

import jax
import jax.numpy as jnp
from jax import lax
from jax.experimental import pallas as pl
from jax.experimental.pallas import tpu as pltpu
import jax.experimental.pallas.tpu_sc as plsc

N = 10000
E = 320000
F = 128
G = 64

NC = 2
NS = 16
NW = NC * NS
EW = E // NW
K = 80
NCHUNK = EW // K
NPAD = 10240
RPT = NPAD // NS


def _aggr_body(x_hbm, src_hbm, dst_hbm, zeros_hbm, out_hbm,
               src_v, dst_v, rows_v, aggr_sh, sem):
    c = lax.axis_index("c")
    s = lax.axis_index("s")
    wid = s * NC + c

    pltpu.sync_copy(zeros_hbm, aggr_sh.at[pl.ds(s * RPT, RPT)])
    plsc.subcore_barrier()

    def chunk(i, carry):
        base = wid * EW + i * K
        pltpu.sync_copy(src_hbm.at[pl.ds(base, K)], src_v)
        pltpu.sync_copy(dst_hbm.at[pl.ds(base, K)], dst_v)
        pltpu.async_copy(x_hbm.at[src_v], rows_v, sem).wait()
        pltpu.sync_copy(rows_v, aggr_sh.at[dst_v], add=True)
        return carry

    lax.fori_loop(0, NCHUNK, chunk, 0)
    plsc.subcore_barrier()

    pltpu.sync_copy(aggr_sh.at[pl.ds(s * RPT, RPT)],
                    out_hbm.at[c, pl.ds(s * RPT, RPT)])


@jax.jit
def _sc_aggregate(x, src, dst, zeros_rows):
    mesh = plsc.VectorSubcoreMesh(core_axis_name="c", subcore_axis_name="s")
    f = pl.kernel(
        _aggr_body,
        out_type=jax.ShapeDtypeStruct((NC, NPAD, F), jnp.float32),
        mesh=mesh,
        scratch_types=[
            pltpu.VMEM((K,), jnp.int32),
            pltpu.VMEM((K,), jnp.int32),
            pltpu.VMEM((K, F), jnp.float32),
            pltpu.VMEM_SHARED((NPAD, F), jnp.float32),
            pltpu.SemaphoreType.DMA,
        ],
    )
    return f(x, src, dst, zeros_rows)


def _layer_tc_body(x_ref, a0_ref, a1_ref, w1_ref, b1_ref, w2_ref, b2_ref,
                   g_ref, be_ref, o_ref):
    h = x_ref[...] + a0_ref[:N] + a1_ref[:N]
    h = jnp.dot(h, w1_ref[...], preferred_element_type=jnp.float32) + b1_ref[...]
    h = jnp.maximum(h, 0.0)
    y = jnp.dot(h, w2_ref[...], preferred_element_type=jnp.float32) + b2_ref[...]
    y = jnp.maximum(y, 0.0)
    mean = jnp.mean(y, axis=0, keepdims=True)
    d = y - mean
    var = jnp.mean(d * d, axis=0, keepdims=True)
    o_ref[...] = d * lax.rsqrt(var + 1e-5) * g_ref[...] + be_ref[...]


@jax.jit
def _layer_tc(x, a0, a1, w1, b1, w2, b2, gamma, beta):
    return pl.pallas_call(
        _layer_tc_body,
        out_shape=jax.ShapeDtypeStruct((N, F), jnp.float32),
    )(x, a0, a1, w1, b1.reshape(1, F), w2, b2.reshape(1, F),
      gamma.reshape(1, F), beta.reshape(1, F))


def _layer3_tc_body(x_ref, a0_ref, a1_ref, w1_ref, b1_ref, w2_ref, b2_ref,
                    g_ref, be_ref, batch_ref, o_ref):
    h = x_ref[...] + a0_ref[:N] + a1_ref[:N]
    h = jnp.dot(h, w1_ref[...], preferred_element_type=jnp.float32) + b1_ref[...]
    h = jnp.maximum(h, 0.0)
    y = jnp.dot(h, w2_ref[...], preferred_element_type=jnp.float32) + b2_ref[...]
    y = jnp.maximum(y, 0.0)
    mean = jnp.mean(y, axis=0, keepdims=True)
    d = y - mean
    var = jnp.mean(d * d, axis=0, keepdims=True)
    hn = d * lax.rsqrt(var + 1e-5) * g_ref[...] + be_ref[...]
    gid = lax.broadcasted_iota(jnp.int32, (G, N), 0)
    onehot = (gid == batch_ref[...]).astype(jnp.float32)
    o_ref[...] = jnp.dot(onehot, hn, preferred_element_type=jnp.float32)


@jax.jit
def _layer3_tc(x, a0, a1, w1, b1, w2, b2, gamma, beta, batch2):
    return pl.pallas_call(
        _layer3_tc_body,
        out_shape=jax.ShapeDtypeStruct((G, F), jnp.float32),
    )(x, a0, a1, w1, b1.reshape(1, F), w2, b2.reshape(1, F),
      gamma.reshape(1, F), beta.reshape(1, F), batch2)


def kernel(x, edge_index, batch, W1_1, b1_1, W1_2, b1_2, gamma1, beta1,
           W2_1, b2_1, W2_2, b2_2, gamma2, beta2,
           W3_1, b3_1, W3_2, b3_2, gamma3, beta3):
    src = edge_index[0]
    dst = edge_index[1]
    zeros_rows = jnp.zeros((RPT, F), dtype=jnp.float32)
    batch2 = batch.reshape(1, N)

    a = _sc_aggregate(x, src, dst, zeros_rows)
    h = _layer_tc(x, a[0], a[1], W1_1, b1_1, W1_2, b1_2, gamma1, beta1)
    a = _sc_aggregate(h, src, dst, zeros_rows)
    h = _layer_tc(h, a[0], a[1], W2_1, b2_1, W2_2, b2_2, gamma2, beta2)
    a = _sc_aggregate(h, src, dst, zeros_rows)
    return _layer3_tc(h, a[0], a[1], W3_1, b3_1, W3_2, b3_2, gamma3, beta3,
                      batch2)


# speedup vs baseline: 10.2372x; 2.2674x over previous
import functools

import jax
import jax.numpy as jnp
from jax import lax
from jax.experimental import pallas as pl
from jax.experimental.pallas import tpu as pltpu
import jax.experimental.pallas.tpu_sc as plsc

N = 10000
E = 320000
F = 128
G = 64

NC = 2
NS = 16
NW = NC * NS
EW = E // NW
K = 80
NCHUNK = EW // K
NPAD = 10240
RPT = NPAD // NS


def _aggr_body(x_hbm, src_hbm, dst_hbm, zeros_hbm, out_hbm,
               src_all, dst_all, rows_v, aggr_sh, sem0, sem1):
    c = lax.axis_index("c")
    s = lax.axis_index("s")
    wid = s * NC + c
    sems = (sem0, sem1)

    pltpu.sync_copy(zeros_hbm, aggr_sh.at[pl.ds(s * RPT, RPT)])

    pltpu.sync_copy(src_hbm.at[pl.ds(wid * EW, EW)], src_all)
    pltpu.sync_copy(dst_hbm.at[wid], dst_all)
    plsc.subcore_barrier()

    def gather(i, b):
        pltpu.async_copy(x_hbm.at[src_all.at[pl.ds(i * K, K)]],
                         rows_v.at[b], sems[b])

    def gather_wait(b):
        pltpu.make_async_copy(x_hbm.at[pl.ds(0, K)], rows_v.at[b],
                              sems[b]).wait()

    gather(0, 0)
    gather(1, 1)

    def pair(j, carry):
        for b in range(2):
            i = 2 * j + b
            gather_wait(b)
            pltpu.sync_copy(rows_v.at[b], aggr_sh.at[dst_all.at[i]],
                            add=True)

            @pl.when(i + 2 < NCHUNK)
            def _():
                gather(i + 2, b)
        return carry

    lax.fori_loop(0, (NCHUNK - 1) // 2, pair, 0)
    i = NCHUNK - 1
    gather_wait(i % 2)
    pltpu.sync_copy(rows_v.at[i % 2], aggr_sh.at[dst_all.at[i]], add=True)

    plsc.subcore_barrier()
    pltpu.sync_copy(aggr_sh.at[pl.ds(s * RPT, RPT)],
                    out_hbm.at[c, pl.ds(s * RPT, RPT)])


@jax.jit
def _sc_aggregate(x, src, dst3, zeros_rows):
    mesh = plsc.VectorSubcoreMesh(core_axis_name="c", subcore_axis_name="s")
    f = pl.kernel(
        _aggr_body,
        out_type=jax.ShapeDtypeStruct((NC, NPAD, F), jnp.float32),
        mesh=mesh,
        scratch_types=[
            pltpu.VMEM((EW,), jnp.int32),
            pltpu.VMEM((NCHUNK, K), jnp.int32),
            pltpu.VMEM((2, K, F), jnp.float32),
            pltpu.VMEM_SHARED((NPAD, F), jnp.float32),
            pltpu.SemaphoreType.DMA,
            pltpu.SemaphoreType.DMA,
        ],
    )
    return f(x, src, dst3, zeros_rows)


def _layer_tc_body(x_ref, a0_ref, a1_ref, w1_ref, b1_ref, w2_ref, b2_ref,
                   g_ref, be_ref, o_ref):
    h = x_ref[...] + a0_ref[:N] + a1_ref[:N]
    h = jnp.dot(h, w1_ref[...], preferred_element_type=jnp.float32) + b1_ref[...]
    h = jnp.maximum(h, 0.0)
    y = jnp.dot(h, w2_ref[...], preferred_element_type=jnp.float32) + b2_ref[...]
    y = jnp.maximum(y, 0.0)
    mean = jnp.mean(y, axis=0, keepdims=True)
    d = y - mean
    var = jnp.mean(d * d, axis=0, keepdims=True)
    o_ref[...] = d * lax.rsqrt(var + 1e-5) * g_ref[...] + be_ref[...]


@jax.jit
def _layer_tc(x, a0, a1, w1, b1, w2, b2, gamma, beta):
    return pl.pallas_call(
        _layer_tc_body,
        out_shape=jax.ShapeDtypeStruct((N, F), jnp.float32),
    )(x, a0, a1, w1, b1.reshape(1, F), w2, b2.reshape(1, F),
      gamma.reshape(1, F), beta.reshape(1, F))


def _layer3_tc_body(x_ref, a0_ref, a1_ref, w1_ref, b1_ref, w2_ref, b2_ref,
                    g_ref, be_ref, batch_ref, o_ref):
    h = x_ref[...] + a0_ref[:N] + a1_ref[:N]
    h = jnp.dot(h, w1_ref[...], preferred_element_type=jnp.float32) + b1_ref[...]
    h = jnp.maximum(h, 0.0)
    y = jnp.dot(h, w2_ref[...], preferred_element_type=jnp.float32) + b2_ref[...]
    y = jnp.maximum(y, 0.0)
    mean = jnp.mean(y, axis=0, keepdims=True)
    d = y - mean
    var = jnp.mean(d * d, axis=0, keepdims=True)
    hn = d * lax.rsqrt(var + 1e-5) * g_ref[...] + be_ref[...]
    gid = lax.broadcasted_iota(jnp.int32, (G, N), 0)
    onehot = (gid == batch_ref[...]).astype(jnp.float32)
    o_ref[...] = jnp.dot(onehot, hn, preferred_element_type=jnp.float32)


@jax.jit
def _layer3_tc(x, a0, a1, w1, b1, w2, b2, gamma, beta, batch2):
    return pl.pallas_call(
        _layer3_tc_body,
        out_shape=jax.ShapeDtypeStruct((G, F), jnp.float32),
    )(x, a0, a1, w1, b1.reshape(1, F), w2, b2.reshape(1, F),
      gamma.reshape(1, F), beta.reshape(1, F), batch2)


def kernel(x, edge_index, batch, W1_1, b1_1, W1_2, b1_2, gamma1, beta1,
           W2_1, b2_1, W2_2, b2_2, gamma2, beta2,
           W3_1, b3_1, W3_2, b3_2, gamma3, beta3):
    src = edge_index[0]
    dst3 = edge_index[1].reshape(NW, NCHUNK, K)
    zeros_rows = jnp.zeros((RPT, F), dtype=jnp.float32)
    batch2 = batch.reshape(1, N)

    a = _sc_aggregate(x, src, dst3, zeros_rows)
    h = _layer_tc(x, a[0], a[1], W1_1, b1_1, W1_2, b1_2, gamma1, beta1)
    a = _sc_aggregate(h, src, dst3, zeros_rows)
    h = _layer_tc(h, a[0], a[1], W2_1, b2_1, W2_2, b2_2, gamma2, beta2)
    a = _sc_aggregate(h, src, dst3, zeros_rows)
    return _layer3_tc(h, a[0], a[1], W3_1, b3_1, W3_2, b3_2, gamma3, beta3,
                      batch2)
